# baseline (speedup 1.0000x reference)
import jax
import jax.numpy as jnp
from jax import lax
from jax.experimental import pallas as pl
from jax.experimental.pallas import tpu as pltpu
from jax.experimental.pallas import tpu_sc as plsc

DIM = 200
SEQ = 16906
COLS = 16896
CW = 2816
NROWCHUNKS = DIM // 8
NCOLCHUNKS = COLS // CW
NCHUNKS = NROWCHUNKS * NCOLCHUNKS
NW = 32


def _sc_body(src_hbm, out_hbm, b0, b1, b2, rs0, rs1, rs2, ws0, ws1, ws2):
    wid = lax.axis_index("s") * 2 + lax.axis_index("c")

    def src_sl(c):
        r = (c % NROWCHUNKS) * 8
        col = (c // NROWCHUNKS) * CW
        return src_hbm.at[pl.ds(r, 8), pl.ds(col, CW)]

    def out_sl(c):
        r = (c % NROWCHUNKS) * 8
        col = (c // NROWCHUNKS) * CW
        return out_hbm.at[pl.ds(r, 8), pl.ds(col, CW)]

    c = [wid + NW * k for k in range(5)]
    has5 = c[4] < NCHUNKS

    @pl.when(has5)
    def _fire5():
        pltpu.async_copy(src_sl(c[4]), b2, rs2)

    r0 = pltpu.async_copy(src_sl(c[0]), b0, rs0)
    r1 = pltpu.async_copy(src_sl(c[1]), b1, rs1)
    r0.wait()
    w0 = pltpu.async_copy(b0, out_sl(c[0]), ws0)
    r1.wait()
    w1 = pltpu.async_copy(b1, out_sl(c[1]), ws1)
    w0.wait()
    r2 = pltpu.async_copy(src_sl(c[2]), b0, rs0)
    r2.wait()
    w2 = pltpu.async_copy(b0, out_sl(c[2]), ws0)
    w1.wait()
    r3 = pltpu.async_copy(src_sl(c[3]), b1, rs1)
    r3.wait()
    w3 = pltpu.async_copy(b1, out_sl(c[3]), ws1)

    @pl.when(has5)
    def _write5():
        pltpu.make_async_copy(src_sl(c[4]), b2, rs2).wait()
        pltpu.async_copy(b2, out_sl(c[4]), ws2)

    w2.wait()
    w3.wait()

    @pl.when(has5)
    def _drain5():
        pltpu.make_async_copy(b2, out_sl(c[4]), ws2).wait()


def _tc_tail_body(part_ref, src_ref, out_ref):
    del part_ref
    out_ref[...] = src_ref[...]


def kernel(x, table):
    del x
    tsrc = jnp.swapaxes(table, 0, 1)
    sc_run = pl.kernel(
        _sc_body,
        out_type=jax.ShapeDtypeStruct((DIM, SEQ), jnp.float32),
        mesh=plsc.VectorSubcoreMesh(core_axis_name="c", subcore_axis_name="s"),
        scratch_types=[pltpu.VMEM((8, CW), jnp.float32),
                       pltpu.VMEM((8, CW), jnp.float32),
                       pltpu.VMEM((8, CW), jnp.float32),
                       pltpu.SemaphoreType.DMA,
                       pltpu.SemaphoreType.DMA,
                       pltpu.SemaphoreType.DMA,
                       pltpu.SemaphoreType.DMA,
                       pltpu.SemaphoreType.DMA,
                       pltpu.SemaphoreType.DMA],
    )
    part = sc_run(tsrc)
    tail = lax.slice(tsrc, (0, COLS), (DIM, SEQ))
    out_t = lax.dynamic_update_slice(part, tail, (0, COLS))
    return jnp.swapaxes(out_t, 0, 1)


# speedup vs baseline: 1.0049x; 1.0049x over previous
import jax
import jax.numpy as jnp
from jax import lax
from jax.experimental import pallas as pl
from jax.experimental.pallas import tpu as pltpu
from jax.experimental.pallas import tpu_sc as plsc

DIM = 200
SEQ = 16906
COLS = 16896
CW = 2816
NROWCHUNKS = DIM // 8
NCOLCHUNKS = COLS // CW
NCHUNKS = NROWCHUNKS * NCOLCHUNKS
NW = 32


def _sc_body(src_hbm, out_hbm, b0, b1, b2, rs0, rs1, rs2, ws0, ws1, ws2):
    wid = lax.axis_index("s") * 2 + lax.axis_index("c")

    def src_sl(c):
        r = (c % NROWCHUNKS) * 8
        col = (c // NROWCHUNKS) * CW
        return src_hbm.at[pl.ds(r, 8), pl.ds(col, CW)]

    def out_sl(c):
        r = (c % NROWCHUNKS) * 8
        col = (c // NROWCHUNKS) * CW
        return out_hbm.at[pl.ds(r, 8), pl.ds(col, CW)]

    c = [wid + NW * k for k in range(5)]
    has5 = c[4] < NCHUNKS

    @pl.when(has5)
    def _fire5():
        pltpu.async_copy(src_sl(c[4]), b2, rs2)

    r0 = pltpu.async_copy(src_sl(c[0]), b0, rs0)
    r1 = pltpu.async_copy(src_sl(c[1]), b1, rs1)
    r0.wait()
    w0 = pltpu.async_copy(b0, out_sl(c[0]), ws0)
    r1.wait()
    w1 = pltpu.async_copy(b1, out_sl(c[1]), ws1)
    w0.wait()
    r2 = pltpu.async_copy(src_sl(c[2]), b0, rs0)
    r2.wait()
    w2 = pltpu.async_copy(b0, out_sl(c[2]), ws0)
    w1.wait()
    r3 = pltpu.async_copy(src_sl(c[3]), b1, rs1)
    r3.wait()
    w3 = pltpu.async_copy(b1, out_sl(c[3]), ws1)

    @pl.when(has5)
    def _write5():
        pltpu.make_async_copy(src_sl(c[4]), b2, rs2).wait()
        pltpu.async_copy(b2, out_sl(c[4]), ws2)

    w2.wait()
    w3.wait()

    @pl.when(has5)
    def _drain5():
        pltpu.make_async_copy(b2, out_sl(c[4]), ws2).wait()


def _tc_tail_body(part_ref, src_ref, out_ref):
    del part_ref
    out_ref[...] = src_ref[...]


def kernel(x, table):
    del x
    tsrc = jnp.swapaxes(table, 0, 1)
    sc_run = pl.kernel(
        _sc_body,
        out_type=jax.ShapeDtypeStruct((DIM, SEQ), jnp.float32),
        mesh=plsc.VectorSubcoreMesh(core_axis_name="c", subcore_axis_name="s"),
        scratch_types=[pltpu.VMEM((8, CW), jnp.float32),
                       pltpu.VMEM((8, CW), jnp.float32),
                       pltpu.VMEM((8, CW), jnp.float32),
                       pltpu.SemaphoreType.DMA,
                       pltpu.SemaphoreType.DMA,
                       pltpu.SemaphoreType.DMA,
                       pltpu.SemaphoreType.DMA,
                       pltpu.SemaphoreType.DMA,
                       pltpu.SemaphoreType.DMA],
    )
    part = sc_run(tsrc)
    out_t = pl.pallas_call(
        _tc_tail_body,
        grid=(1,),
        in_specs=[
            pl.BlockSpec((8, 128), lambda i: (0, COLS // 128)),
            pl.BlockSpec((DIM, 128), lambda i: (0, COLS // 128)),
        ],
        out_specs=pl.BlockSpec((DIM, 128), lambda i: (0, COLS // 128)),
        out_shape=jax.ShapeDtypeStruct((DIM, SEQ), jnp.float32),
        input_output_aliases={0: 0},
    )(part, tsrc)
    return jnp.swapaxes(out_t, 0, 1)
